# baseline (speedup 1.0000x reference)
import jax
import jax.numpy as jnp
import numpy as np
from jax.experimental import pallas as pl
from jax.experimental.pallas import tpu as pltpu

_BLOCK = 4000
_G = 25


def _fused_kernel(op_hbm, tb_hbm, ft_hbm, jn_hbm, cd_hbm,
                  w1op_ref, w1tb_ref, w1ft_ref, w1jn_ref, w1cd_ref, b1_ref,
                  w2op_ref, w2tb_ref, w2ft_ref, w2jn_ref, w2cd_ref, b2_ref,
                  wx_ref, bx_ref, wo1_ref, bo1_ref, wo2_ref, bo2_ref,
                  out_ref, c_ref,
                  op_v, tb_v, ft_v, jn_v, cd_v, sems):
    i = pl.program_id(0)
    b = _BLOCK
    srcs = (op_hbm, tb_hbm, ft_hbm, jn_hbm, cd_hbm)
    bufs = (op_v, tb_v, ft_v, jn_v, cd_v)
    slot = jax.lax.rem(i, 2)
    nslot = jax.lax.rem(i + 1, 2)

    def start_block(blk_idx, slot_idx):
        for k in range(5):
            pltpu.make_async_copy(
                srcs[k].at[pl.ds(blk_idx * b, b), :],
                bufs[k].at[slot_idx],
                sems.at[slot_idx, k]).start()

    @pl.when(i == 0)
    def _():
        start_block(0, 0)

    @pl.when(i + 1 < pl.num_programs(0))
    def _():
        start_block(i + 1, nslot)

    for k in range(5):
        pltpu.make_async_copy(
            srcs[k].at[pl.ds(i * b, b), :],
            bufs[k].at[slot],
            sems.at[slot, k]).wait()

    relu = jax.nn.relu

    def dot(a, bb):
        return jnp.dot(a, bb, preferred_element_type=jnp.float32)

    b1 = b1_ref[...]
    h_op = relu(dot(op_v[slot], w1op_ref[...]) + b1[:, 0:16])
    h_tb = relu(dot(tb_v[slot], w1tb_ref[...]) + b1[:, 16:32])
    h_ft = relu(dot(ft_v[slot], w1ft_ref[...]) + b1[:, 32:48])
    h_jn = relu(dot(jn_v[slot], w1jn_ref[...]) + b1[:, 48:64])
    h_cd = relu(dot(cd_v[slot], w1cd_ref[...]) + b1[:, 64:80])
    x = relu(dot(h_op, w2op_ref[...]) + dot(h_tb, w2tb_ref[...])
             + dot(h_ft, w2ft_ref[...]) + dot(h_jn, w2jn_ref[...])
             + dot(h_cd, w2cd_ref[...]) + b2_ref[...])
    xou = dot(x, wx_ref[...]) + bx_ref[...]
    xx = xou[:, 0:80]
    ff = jax.nn.sigmoid(xou[:, 128:208])
    rr = jax.nn.sigmoid(xou[:, 256:336])
    c = (1.0 - ff) * xx
    h = rr * jnp.tanh(c) + (1.0 - rr) * x
    hid = relu(dot(h, wo1_ref[...]) + bo1_ref[...])
    out_ref[...] = jax.nn.sigmoid(dot(hid, wo2_ref[...]) + bo2_ref[...])
    c_ref[...] = c


@jax.jit
def _run(op_feat, tb_feat, ft_feat, join_feat, card_feat,
         w1op, w1tb, w1ft, w1jn, w1cd, b1,
         w2op, w2tb, w2ft, w2jn, w2cd, b2,
         wx, bx, wo1, bo1, wo2, bo2):
    n = op_feat.shape[0]
    blk = _BLOCK
    grid = (n // blk,)

    def rows(i):
        return (i, 0)

    def whole(i):
        return (0, 0)

    any_spec = pl.BlockSpec(memory_space=pl.ANY)
    row_spec = lambda w: pl.BlockSpec((blk, w), rows)
    full_spec = lambda a, b: pl.BlockSpec((a, b), whole)

    out, c = pl.pallas_call(
        _fused_kernel,
        grid=grid,
        in_specs=[
            any_spec, any_spec, any_spec, any_spec, any_spec,
            full_spec(16, 16), full_spec(32, 16), full_spec(64, 16),
            full_spec(32, 16), full_spec(16, 16), full_spec(1, 80),
            full_spec(16, 80), full_spec(16, 80), full_spec(16, 80),
            full_spec(16, 80), full_spec(16, 80), full_spec(1, 80),
            full_spec(80, 384), full_spec(1, 384),
            full_spec(80, 64), full_spec(1, 64),
            full_spec(64, 1), full_spec(1, 1),
        ],
        out_specs=[row_spec(1), row_spec(80)],
        out_shape=[
            jax.ShapeDtypeStruct((n, 1), jnp.float32),
            jax.ShapeDtypeStruct((n, 80), jnp.float32),
        ],
        scratch_shapes=[
            pltpu.VMEM((2, blk, 16), jnp.float32),
            pltpu.VMEM((2, blk, 32), jnp.float32),
            pltpu.VMEM((2, blk, 64), jnp.float32),
            pltpu.VMEM((2, blk, 32), jnp.float32),
            pltpu.VMEM((2, blk, 16), jnp.float32),
            pltpu.SemaphoreType.DMA((2, 5)),
        ],
    )(op_feat, tb_feat, ft_feat, join_feat, card_feat,
      w1op, w1tb, w1ft, w1jn, w1cd, b1,
      w2op, w2tb, w2ft, w2jn, w2cd, b2,
      wx, bx, wo1, bo1, wo2, bo2)
    return out, c


def _place(w, col):
    out = jnp.zeros((16, 80), jnp.float32)
    return out.at[:, col:col + 16].set(w)


def kernel(op_feat, tb_feat, ft_feat, join_feat, card_feat, node_order,
           adjacency_list, edge_order,
           W_op, b_op, W_op2, b_op2, W_tb, b_tb, W_tb2, b_tb2,
           W_ft, b_ft, W_ft2, b_ft2, W_jn, b_jn, W_jn2, b_jn2,
           W_cd, b_cd, W_cd2, b_cd2, W_xou, b_xou, W_o1, b_o1, W_o2, b_o2):
    b1 = jnp.concatenate([b_op, b_tb, b_ft, b_jn, b_cd])[None, :]
    b2 = jnp.concatenate([b_op2, b_tb2, b_ft2, b_jn2, b_cd2])[None, :]
    wxT = W_xou.T
    wx = jnp.zeros((80, 384), jnp.float32)
    wx = wx.at[:, 0:80].set(wxT[:, 0:80])
    wx = wx.at[:, 128:208].set(wxT[:, 80:160])
    wx = wx.at[:, 256:336].set(wxT[:, 160:240])
    bx = jnp.zeros((1, 384), jnp.float32)
    bx = bx.at[0, 0:80].set(b_xou[0:80])
    bx = bx.at[0, 128:208].set(b_xou[80:160])
    bx = bx.at[0, 256:336].set(b_xou[160:240])
    return _run(
        op_feat, tb_feat, ft_feat, join_feat, card_feat,
        W_op.T, W_tb.T, W_ft.T, W_jn.T, W_cd.T, b1,
        _place(W_op2.T, 0), _place(W_tb2.T, 16), _place(W_ft2.T, 32),
        _place(W_jn2.T, 48), _place(W_cd2.T, 64), b2,
        wx, bx, W_o1.T, b_o1[None, :], W_o2.T, b_o2[None, :])
